# baseline (speedup 1.0000x reference)
import functools

import jax
import jax.numpy as jnp
from jax import lax
from jax.experimental import pallas as pl
from jax.experimental.pallas import tpu as pltpu
from jax.experimental.pallas import tpu_sc as plsc

N = 10000
E = 320000
D = 128
H = 8
DH = 16
W144 = 144
INV_BN = 0.9999950000374997

NW = 32
EPW = E // NW
CH = 16
ESUP = 2000
CSUP = ESUP // CH
ZBASE = 10000
ACC_ROWS = 10752
RPS = ACC_ROWS // 16


def _qkv_body(h_ref, wq_ref, wkv_ref, q_ref, kv_ref):
    hb = h_ref[...]
    q_ref[...] = (jnp.dot(hb, wq_ref[...],
                          preferred_element_type=jnp.float32)
                  * 0.25).astype(jnp.bfloat16)
    kv_ref[...] = jnp.dot(
        hb, wkv_ref[...],
        preferred_element_type=jnp.float32).astype(jnp.bfloat16)


def _run_qkv(h, wqp, wkv):
    blk = 1000
    return pl.pallas_call(
        _qkv_body,
        grid=(N // blk,),
        in_specs=[
            pl.BlockSpec((blk, D), lambda i: (i, 0)),
            pl.BlockSpec((D, 2 * D), lambda i: (0, 0)),
            pl.BlockSpec((D, 2 * D), lambda i: (0, 0)),
        ],
        out_specs=[
            pl.BlockSpec((blk, 2 * D), lambda i: (i, 0)),
            pl.BlockSpec((blk, 2 * D), lambda i: (i, 0)),
        ],
        out_shape=[
            jax.ShapeDtypeStruct((N, 2 * D), jnp.bfloat16),
            jax.ShapeDtypeStruct((N, 2 * D), jnp.bfloat16),
        ],
    )(h, wqp, wkv)


def _ee_body(ea_ref, w_ref, b_ref, out_ref):
    out_ref[...] = (
        jnp.dot(ea_ref[...], w_ref[...], preferred_element_type=jnp.float32)
        + b_ref[...]
    )


def _run_ee(edge_attr, wcat, bcat):
    blk = 4000
    return pl.pallas_call(
        _ee_body,
        grid=(E // blk,),
        in_specs=[
            pl.BlockSpec((blk, 16), lambda i: (i, 0)),
            pl.BlockSpec((16, W144), lambda i: (0, 0)),
            pl.BlockSpec((1, W144), lambda i: (0, 0)),
        ],
        out_specs=pl.BlockSpec((blk, W144), lambda i: (i, 0)),
        out_shape=jax.ShapeDtypeStruct((E, W144), jnp.float32),
    )(edge_attr, wcat, bcat)


@functools.partial(
    pl.kernel,
    out_type=jax.ShapeDtypeStruct((2, ACC_ROWS, D), jnp.float32),
    mesh=plsc.VectorSubcoreMesh(core_axis_name="c", subcore_axis_name="s"),
    compiler_params=pltpu.CompilerParams(needs_layout_passes=False),
    scratch_types=[
        pltpu.VMEM_SHARED((ACC_ROWS, D), jnp.float32),
        pltpu.VMEM((ESUP,), jnp.int32),
        pltpu.VMEM((ESUP,), jnp.int32),
        pltpu.VMEM((CH, D), jnp.int32),
        pltpu.VMEM((CH, D), jnp.int32),
        pltpu.VMEM((CH, D), jnp.int32),
        pltpu.VMEM((CH, D), jnp.int32),
        pltpu.VMEM((CH, W144), jnp.float32),
        pltpu.VMEM((CH, W144), jnp.float32),
        pltpu.VMEM((CH, D), jnp.float32),
        pltpu.VMEM((CH, D), jnp.float32),
        pltpu.VMEM((CH + 8, D), jnp.float32),
        pltpu.VMEM((CH + 8, D), jnp.float32),
        pltpu.VMEM((CH,), jnp.int32),
        pltpu.VMEM((CH,), jnp.int32),
        pltpu.VMEM((CH,), jnp.int32),
        pltpu.VMEM((CH,), jnp.int32),
        pltpu.VMEM((CH,), jnp.int32),
        pltpu.VMEM((CH,), jnp.int32),
        pltpu.SemaphoreType.DMA,
        pltpu.SemaphoreType.DMA,
        pltpu.SemaphoreType.DMA,
        pltpu.SemaphoreType.DMA,
    ],
)
def _sc_edge(kv_hbm, q_hbm, ee_hbm, src_hbm, dst_hbm, out_hbm,
             acc, srcB, dstB, kvb0, kvb1, qb0, qb1, eeb0, eeb1,
             ob0, ob1, ob20, ob21, dstvS0, dstvS1, dstzvS0, dstzvS1,
             pc0, pc1, semg0, semg1, sems0, sems1):
    c = lax.axis_index("c")
    s = lax.axis_index("s")
    wid = c * 16 + s
    lane = lax.iota(jnp.int32, 16)
    zeros16 = jnp.zeros((16,), jnp.float32)
    pm4 = jnp.bitwise_xor(lane, 4)
    pm2 = jnp.bitwise_xor(lane, 2)
    pm1 = jnp.bitwise_xor(lane, 1)
    m4 = jnp.bitwise_and(lane, 4) == 0
    m2 = jnp.bitwise_and(lane, 2) == 0
    fl8 = lax.shift_right_logical(lane, 3)
    pfin = jnp.bitwise_or(
        jnp.bitwise_or(lax.shift_left(jnp.bitwise_and(lane, 1), 3),
                       lax.shift_left(jnp.bitwise_and(lane, 2), 1)),
        lax.shift_right_logical(jnp.bitwise_and(lane, 4), 1))

    def _gx(v, pm):
        return v.at[pm].get(mode="promise_in_bounds")

    def _unpk(w):
        ev = plsc.bitcast(lax.shift_left(w, 16), jnp.float32)
        od = plsc.bitcast(
            jnp.bitwise_and(w, jnp.int32(-65536)), jnp.float32)
        return ev, od

    def _zrow(r, carry):
        for j in range(D // 16):
            ob0[r, pl.ds(16 * j, 16)] = zeros16
            ob1[r, pl.ds(16 * j, 16)] = zeros16
            ob20[r, pl.ds(16 * j, 16)] = zeros16
            ob21[r, pl.ds(16 * j, 16)] = zeros16
        return carry

    lax.fori_loop(0, CH, _zrow, 0)
    for j in range(RPS // CH):
        pltpu.sync_copy(ob0, acc.at[pl.ds(s * RPS + j * CH, CH)])
    plsc.subcore_barrier()

    ebase = wid * EPW

    gsets = ((kvb0, qb0, eeb0, semg0), (kvb1, qb1, eeb1, semg1))
    osets = ((ob0, ob20, dstvS0, dstzvS0, pc0, sems0),
             (ob1, ob21, dstvS1, dstzvS1, pc1, sems1))

    def _issue_scatter(ob, ob2, dstvS, dstzvS, pcS, sems):
        pltpu.async_copy(ob, acc.at[dstvS], sems, add=True)
        pltpu.async_copy(ob2.at[pl.ds(0, CH)], acc.at[dstzvS], sems,
                         add=True)

    def _wait_scatter(ob, ob2, dstvS, dstzvS, pcS, sems):
        pltpu.make_async_copy(ob, acc.at[dstvS], sems).wait()
        pltpu.make_async_copy(ob2.at[pl.ds(0, CH)], acc.at[dstzvS],
                              sems).wait()

    for (ob, ob2, dstvS, dstzvS, pcS, sems) in osets:
        dstvS[...] = lane
        dstzvS[...] = lane
        pcS[...] = jnp.bitwise_and(lane, 0)
        _issue_scatter(ob, ob2, dstvS, dstzvS, pcS, sems)

    def _issue(ci, sbase, kvb, qb, eeb, semg):
        pltpu.async_copy(kv_hbm.at[srcB.at[pl.ds(ci * CH, CH)]], kvb, semg)
        pltpu.async_copy(q_hbm.at[dstB.at[pl.ds(ci * CH, CH)]], qb, semg)
        pltpu.async_copy(ee_hbm.at[pl.ds(sbase + ci * CH, CH)], eeb, semg)

    def _drain(kvb, qb, eeb, semg):
        pltpu.make_async_copy(kv_hbm.at[srcB.at[pl.ds(0, CH)]], kvb,
                              semg).wait()
        pltpu.make_async_copy(q_hbm.at[dstB.at[pl.ds(0, CH)]], qb,
                              semg).wait()
        pltpu.make_async_copy(ee_hbm.at[pl.ds(0, CH)], eeb, semg).wait()

    def _compute_scatter(ci, kvb, qb, eeb, ob, ob2, dstvS, dstzvS, pcS,
                         sems):
        _wait_scatter(ob, ob2, dstvS, dstzvS, pcS, sems)
        dwin = dstB[pl.ds(ci * CH, 16)]
        dstvS[...] = dwin
        dstzvS[...] = ZBASE + lax.shift_right_logical(dwin, 4)
        c0vec = lax.mul(jnp.bitwise_and(dwin, 15), 8)
        pold = pcS[...]
        pcS[...] = c0vec
        for j in range(16):
            e = j
            ts = []
            for hp in range(4):
                kE, kO = _unpk(kvb[e, pl.ds(16 * hp, 16)])
                qE, qO = _unpk(qb[e, pl.ds(16 * hp, 16)])
                t = (kE * qE * eeb[e, pl.ds(32 * hp, 16)]
                     + kO * qO * eeb[e, pl.ds(32 * hp + 16, 16)])
                t = t + _gx(t, pm4)
                ts.append(t)
            u03 = jnp.where(m4, ts[0], ts[1])
            u47 = jnp.where(m4, ts[2], ts[3])
            u03 = u03 + _gx(u03, pm2)
            u47 = u47 + _gx(u47, pm2)
            f = jnp.where(m2, u03, u47)
            f = f + _gx(f, pm1)
            sv = _gx(f, pfin)
            eb = eeb[e, pl.ds(128, 16)]
            sco = jnp.exp(jnp.clip(sv + eb, -5.0, 5.0))
            for hp in range(4):
                vE, vO = _unpk(kvb[e, pl.ds(64 + 16 * hp, 16)])
                shv = _gx(sco, 2 * hp + fl8)
                ob[e, pl.ds(32 * hp, 16)] = vE * shv
                ob[e, pl.ds(32 * hp + 16, 16)] = vO * shv
            scoz = jnp.where(lane < H, sco, 0.0)
            ob2[e, pl.ds(pold[j], 16)] = zeros16
            ob2[e, pl.ds(c0vec[j], 16)] = scoz
        _issue_scatter(ob, ob2, dstvS, dstzvS, pcS, sems)

    def _super(u, carry):
        sbase = ebase + u * ESUP
        pltpu.sync_copy(src_hbm.at[pl.ds(sbase, ESUP)], srcB)
        pltpu.sync_copy(dst_hbm.at[pl.ds(sbase, ESUP)], dstB)
        _issue(0, sbase, *gsets[0])
        _issue(1, sbase, *gsets[1])

        def _pair(pp, pcarry):
            for b in range(2):
                kvb, qb, eeb, semg = gsets[b]
                ci = 2 * pp + b
                _drain(kvb, qb, eeb, semg)
                _compute_scatter(ci, kvb, qb, eeb, *osets[b])
                cn = jnp.minimum(ci + 2, CSUP - 1)
                _issue(cn, sbase, kvb, qb, eeb, semg)
            return pcarry

        lax.fori_loop(0, (CSUP - 1) // 2, _pair, 0)
        _drain(*gsets[0])
        _compute_scatter(CSUP - 1, *gsets[0][:3], *osets[0])
        _drain(*gsets[1])
        return carry

    lax.fori_loop(0, EPW // ESUP, _super, 0)

    for (ob, ob2, dstvS, dstzvS, pcS, sems) in osets:
        _wait_scatter(ob, ob2, dstvS, dstzvS, pcS, sems)
    plsc.subcore_barrier()
    pltpu.sync_copy(acc.at[pl.ds(s * RPS, RPS)],
                    out_hbm.at[c, pl.ds(s * RPS, RPS)])


def _post_body(p_ref, z_ref, h_ref, pt_ref, sel_ref, g1_ref, be1_ref,
               wf1_ref, bf1_ref, wf2_ref, bf2_ref, g2_ref, be2_ref,
               out_ref):
    p = p_ref[...]
    wv = jnp.dot(p[0] + p[1], pt_ref[...],
                 preferred_element_type=jnp.float32)
    z = z_ref[...]
    z8 = z[0] + z[1]
    zr = jnp.dot(z8, sel_ref[...], preferred_element_type=jnp.float32)
    ha = h_ref[...] + wv / (zr + 1e-6)
    hn = ha * (g1_ref[...] * INV_BN) + be1_ref[...]
    ff = jnp.maximum(
        jnp.dot(hn, wf1_ref[...], preferred_element_type=jnp.float32)
        + bf1_ref[...], 0.0)
    ff = jnp.dot(ff, wf2_ref[...], preferred_element_type=jnp.float32) + bf2_ref[...]
    out_ref[...] = (hn + ff) * (g2_ref[...] * INV_BN) + be2_ref[...]


def _run_post(pacc, z, h, pt, sel, g1, be1, wf1, bf1, wf2, bf2, g2, be2):
    blk = 1000
    full = lambda shape: pl.BlockSpec(shape, lambda i: tuple(0 for _ in shape))
    return pl.pallas_call(
        _post_body,
        grid=(N // blk,),
        in_specs=[
            pl.BlockSpec((2, blk, D), lambda i: (0, i, 0)),
            pl.BlockSpec((2, blk, H), lambda i: (0, i, 0)),
            pl.BlockSpec((blk, D), lambda i: (i, 0)),
            full((D, D)),
            full((H, D)),
            full((1, D)),
            full((1, D)),
            full((D, 2 * D)),
            full((1, 2 * D)),
            full((2 * D, D)),
            full((1, D)),
            full((1, D)),
            full((1, D)),
        ],
        out_specs=pl.BlockSpec((blk, D), lambda i: (i, 0)),
        out_shape=jax.ShapeDtypeStruct((N, D), jnp.float32),
    )(pacc, z, h, pt, sel, g1, be1, wf1, bf1, wf2, bf2, g2, be2)


def kernel(h, edge_index, edge_attr, WQ, WK, WV, WE, WEb, bEb, g1, be1,
           Wf1, bf1, Wf2, bf2, g2, be2):
    src = edge_index[0].astype(jnp.int32)
    dst = edge_index[1].astype(jnp.int32)

    p = jnp.arange(D, dtype=jnp.int32)
    r = p % 32
    colperm = 32 * (p // 32) + jnp.where(r < 16, 2 * r, 2 * (r - 16) + 1)
    pt = (jnp.arange(D)[None, :] == colperm[:, None]).astype(jnp.float32)

    wqp = jnp.concatenate([WQ, jnp.zeros((D, D), jnp.float32)], axis=1)
    wkv = jnp.concatenate([WK, WV], axis=1)
    wcat = jnp.concatenate(
        [WE[:, colperm], WEb, jnp.zeros((16, 8), jnp.float32)], axis=1)
    bcat = jnp.concatenate(
        [jnp.zeros((D,), jnp.float32), bEb, jnp.zeros((8,), jnp.float32)]
    ).reshape(1, W144)

    q, kv = _run_qkv(h, wqp, wkv)
    eeeb = _run_ee(edge_attr, wcat, bcat)
    kv32 = lax.bitcast_convert_type(kv.reshape(N, D, 2), jnp.int32)
    q32 = lax.bitcast_convert_type(q.reshape(N, D, 2), jnp.int32)
    pacc = _sc_edge(kv32, q32, eeeb, src, dst)

    z = pacc[:, ZBASE:ZBASE + (N * H) // D, :].reshape(2, N, H)

    sel = jnp.kron(jnp.eye(H, dtype=jnp.float32),
                   jnp.ones((1, DH), jnp.float32))
    h_out = _run_post(pacc, z, h, pt, sel,
                      g1.reshape(1, D), be1.reshape(1, D),
                      Wf1, bf1.reshape(1, 2 * D),
                      Wf2, bf2.reshape(1, D),
                      g2.reshape(1, D), be2.reshape(1, D))
    return (h_out, edge_attr)


# speedup vs baseline: 1.0005x; 1.0005x over previous
import functools

import jax
import jax.numpy as jnp
from jax import lax
from jax.experimental import pallas as pl
from jax.experimental.pallas import tpu as pltpu
from jax.experimental.pallas import tpu_sc as plsc

N = 10000
E = 320000
D = 128
H = 8
DH = 16
W144 = 144
INV_BN = 0.9999950000374997

NW = 32
EPW = E // NW
CH = 16
ESUP = 2000
CSUP = ESUP // CH
ZBASE = 10000
ACC_ROWS = 10752
RPS = ACC_ROWS // 16


def _qkv_body(h_ref, wq_ref, wkv_ref, q_ref, kv_ref):
    hb = h_ref[...]
    q_ref[...] = (jnp.dot(hb, wq_ref[...],
                          preferred_element_type=jnp.float32)
                  * 0.25).astype(jnp.bfloat16)
    kv_ref[...] = jnp.dot(
        hb, wkv_ref[...],
        preferred_element_type=jnp.float32).astype(jnp.bfloat16)


def _run_qkv(h, wqp, wkv):
    blk = 1000
    return pl.pallas_call(
        _qkv_body,
        grid=(N // blk,),
        in_specs=[
            pl.BlockSpec((blk, D), lambda i: (i, 0)),
            pl.BlockSpec((D, 2 * D), lambda i: (0, 0)),
            pl.BlockSpec((D, 2 * D), lambda i: (0, 0)),
        ],
        out_specs=[
            pl.BlockSpec((blk, 2 * D), lambda i: (i, 0)),
            pl.BlockSpec((blk, 2 * D), lambda i: (i, 0)),
        ],
        out_shape=[
            jax.ShapeDtypeStruct((N, 2 * D), jnp.bfloat16),
            jax.ShapeDtypeStruct((N, 2 * D), jnp.bfloat16),
        ],
    )(h, wqp, wkv)


def _ee_body(ea_ref, w_ref, b_ref, out_ref):
    out_ref[...] = (
        jnp.dot(ea_ref[...], w_ref[...], preferred_element_type=jnp.float32)
        + b_ref[...]
    )


def _run_ee(edge_attr, wcat, bcat):
    blk = 4000
    return pl.pallas_call(
        _ee_body,
        grid=(E // blk,),
        in_specs=[
            pl.BlockSpec((blk, 16), lambda i: (i, 0)),
            pl.BlockSpec((16, W144), lambda i: (0, 0)),
            pl.BlockSpec((1, W144), lambda i: (0, 0)),
        ],
        out_specs=pl.BlockSpec((blk, W144), lambda i: (i, 0)),
        out_shape=jax.ShapeDtypeStruct((E, W144), jnp.float32),
    )(edge_attr, wcat, bcat)


@functools.partial(
    pl.kernel,
    out_type=jax.ShapeDtypeStruct((2, ACC_ROWS, D), jnp.float32),
    mesh=plsc.VectorSubcoreMesh(core_axis_name="c", subcore_axis_name="s"),
    compiler_params=pltpu.CompilerParams(needs_layout_passes=False),
    scratch_types=[
        pltpu.VMEM_SHARED((ACC_ROWS, D), jnp.float32),
        pltpu.VMEM((ESUP,), jnp.int32),
        pltpu.VMEM((ESUP,), jnp.int32),
        pltpu.VMEM((CH, D), jnp.int32),
        pltpu.VMEM((CH, D), jnp.int32),
        pltpu.VMEM((CH, D), jnp.int32),
        pltpu.VMEM((CH, D), jnp.int32),
        pltpu.VMEM((CH, W144), jnp.float32),
        pltpu.VMEM((CH, W144), jnp.float32),
        pltpu.VMEM((CH, D), jnp.float32),
        pltpu.VMEM((CH, D), jnp.float32),
        pltpu.VMEM((CH + 8, D), jnp.float32),
        pltpu.VMEM((CH + 8, D), jnp.float32),
        pltpu.VMEM((CH,), jnp.int32),
        pltpu.VMEM((CH,), jnp.int32),
        pltpu.VMEM((CH,), jnp.int32),
        pltpu.VMEM((CH,), jnp.int32),
        pltpu.VMEM((CH,), jnp.int32),
        pltpu.VMEM((CH,), jnp.int32),
        pltpu.SemaphoreType.DMA,
        pltpu.SemaphoreType.DMA,
        pltpu.SemaphoreType.DMA,
        pltpu.SemaphoreType.DMA,
    ],
)
def _sc_edge(kv_hbm, q_hbm, ee_hbm, src_hbm, dst_hbm, out_hbm,
             acc, srcB, dstB, kvb0, kvb1, qb0, qb1, eeb0, eeb1,
             ob0, ob1, ob20, ob21, dstvS0, dstvS1, dstzvS0, dstzvS1,
             pc0, pc1, semg0, semg1, sems0, sems1):
    c = lax.axis_index("c")
    s = lax.axis_index("s")
    wid = c * 16 + s
    lane = lax.iota(jnp.int32, 16)
    zeros16 = jnp.zeros((16,), jnp.float32)
    pm4 = jnp.bitwise_xor(lane, 4)
    pm2 = jnp.bitwise_xor(lane, 2)
    pm1 = jnp.bitwise_xor(lane, 1)
    m4 = jnp.bitwise_and(lane, 4) == 0
    m2 = jnp.bitwise_and(lane, 2) == 0
    fl8 = lax.shift_right_logical(lane, 3)
    pfin = jnp.bitwise_or(
        jnp.bitwise_or(lax.shift_left(jnp.bitwise_and(lane, 1), 3),
                       lax.shift_left(jnp.bitwise_and(lane, 2), 1)),
        lax.shift_right_logical(jnp.bitwise_and(lane, 4), 1))

    def _gx(v, pm):
        return v.at[pm].get(mode="promise_in_bounds")

    def _unpk(w):
        ev = plsc.bitcast(lax.shift_left(w, 16), jnp.float32)
        od = plsc.bitcast(
            jnp.bitwise_and(w, jnp.int32(-65536)), jnp.float32)
        return ev, od

    def _zrow(r, carry):
        for j in range(D // 16):
            ob0[r, pl.ds(16 * j, 16)] = zeros16
            ob1[r, pl.ds(16 * j, 16)] = zeros16
            ob20[r, pl.ds(16 * j, 16)] = zeros16
            ob21[r, pl.ds(16 * j, 16)] = zeros16
        return carry

    lax.fori_loop(0, CH, _zrow, 0)
    for j in range(RPS // CH):
        pltpu.sync_copy(ob0, acc.at[pl.ds(s * RPS + j * CH, CH)])
    plsc.subcore_barrier()

    ebase = wid * EPW

    gsets = ((kvb0, qb0, eeb0, semg0), (kvb1, qb1, eeb1, semg1))
    osets = ((ob0, ob20, dstvS0, dstzvS0, pc0, sems0),
             (ob1, ob21, dstvS1, dstzvS1, pc1, sems1))

    def _issue_scatter(ob, ob2, dstvS, dstzvS, pcS, sems):
        pltpu.async_copy(ob, acc.at[dstvS], sems, add=True)
        pltpu.async_copy(ob2.at[pl.ds(0, CH)], acc.at[dstzvS], sems,
                         add=True)

    def _wait_scatter(ob, ob2, dstvS, dstzvS, pcS, sems):
        pltpu.make_async_copy(ob, acc.at[dstvS], sems).wait()
        pltpu.make_async_copy(ob2.at[pl.ds(0, CH)], acc.at[dstzvS],
                              sems).wait()

    for (ob, ob2, dstvS, dstzvS, pcS, sems) in osets:
        dstvS[...] = lane
        dstzvS[...] = lane
        pcS[...] = jnp.bitwise_and(lane, 0)
        _issue_scatter(ob, ob2, dstvS, dstzvS, pcS, sems)

    def _issue(ci, sbase, kvb, qb, eeb, semg):
        pltpu.async_copy(kv_hbm.at[srcB.at[pl.ds(ci * CH, CH)]], kvb, semg)
        pltpu.async_copy(q_hbm.at[dstB.at[pl.ds(ci * CH, CH)]], qb, semg)
        pltpu.async_copy(ee_hbm.at[pl.ds(sbase + ci * CH, CH)], eeb, semg)

    def _drain(kvb, qb, eeb, semg):
        pltpu.make_async_copy(kv_hbm.at[srcB.at[pl.ds(0, CH)]], kvb,
                              semg).wait()
        pltpu.make_async_copy(q_hbm.at[dstB.at[pl.ds(0, CH)]], qb,
                              semg).wait()
        pltpu.make_async_copy(ee_hbm.at[pl.ds(0, CH)], eeb, semg).wait()

    def _compute_scatter(ci, kvb, qb, eeb, ob, ob2, dstvS, dstzvS, pcS,
                         sems):
        _wait_scatter(ob, ob2, dstvS, dstzvS, pcS, sems)
        dwin = dstB[pl.ds(ci * CH, 16)]
        dstvS[...] = dwin
        dstzvS[...] = ZBASE + lax.shift_right_logical(dwin, 4)
        c0vec = lax.mul(jnp.bitwise_and(dwin, 15), 8)
        pold = pcS[...]
        pcS[...] = c0vec
        for j in range(16):
            e = j
            ts = []
            for hp in range(4):
                kq = plsc.bitcast(
                    plsc.bitcast(kvb[e, pl.ds(16 * hp, 16)],
                                 jnp.bfloat16)
                    * plsc.bitcast(qb[e, pl.ds(16 * hp, 16)],
                                   jnp.bfloat16),
                    jnp.int32)
                kqE, kqO = _unpk(kq)
                t = (kqE * eeb[e, pl.ds(32 * hp, 16)]
                     + kqO * eeb[e, pl.ds(32 * hp + 16, 16)])
                t = t + _gx(t, pm4)
                ts.append(t)
            u03 = jnp.where(m4, ts[0], ts[1])
            u47 = jnp.where(m4, ts[2], ts[3])
            u03 = u03 + _gx(u03, pm2)
            u47 = u47 + _gx(u47, pm2)
            f = jnp.where(m2, u03, u47)
            f = f + _gx(f, pm1)
            sv = _gx(f, pfin)
            eb = eeb[e, pl.ds(128, 16)]
            sco = jnp.exp(jnp.clip(sv + eb, -5.0, 5.0))
            for hp in range(4):
                vE, vO = _unpk(kvb[e, pl.ds(64 + 16 * hp, 16)])
                shv = _gx(sco, 2 * hp + fl8)
                ob[e, pl.ds(32 * hp, 16)] = vE * shv
                ob[e, pl.ds(32 * hp + 16, 16)] = vO * shv
            scoz = jnp.where(lane < H, sco, 0.0)
            ob2[e, pl.ds(pold[j], 16)] = zeros16
            ob2[e, pl.ds(c0vec[j], 16)] = scoz
        _issue_scatter(ob, ob2, dstvS, dstzvS, pcS, sems)

    def _super(u, carry):
        sbase = ebase + u * ESUP
        pltpu.sync_copy(src_hbm.at[pl.ds(sbase, ESUP)], srcB)
        pltpu.sync_copy(dst_hbm.at[pl.ds(sbase, ESUP)], dstB)
        _issue(0, sbase, *gsets[0])
        _issue(1, sbase, *gsets[1])

        def _pair(pp, pcarry):
            for b in range(2):
                kvb, qb, eeb, semg = gsets[b]
                ci = 2 * pp + b
                _drain(kvb, qb, eeb, semg)
                _compute_scatter(ci, kvb, qb, eeb, *osets[b])
                cn = jnp.minimum(ci + 2, CSUP - 1)
                _issue(cn, sbase, kvb, qb, eeb, semg)
            return pcarry

        lax.fori_loop(0, (CSUP - 1) // 2, _pair, 0)
        _drain(*gsets[0])
        _compute_scatter(CSUP - 1, *gsets[0][:3], *osets[0])
        _drain(*gsets[1])
        return carry

    lax.fori_loop(0, EPW // ESUP, _super, 0)

    for (ob, ob2, dstvS, dstzvS, pcS, sems) in osets:
        _wait_scatter(ob, ob2, dstvS, dstzvS, pcS, sems)
    plsc.subcore_barrier()
    pltpu.sync_copy(acc.at[pl.ds(s * RPS, RPS)],
                    out_hbm.at[c, pl.ds(s * RPS, RPS)])


def _post_body(p_ref, z_ref, h_ref, pt_ref, sel_ref, g1_ref, be1_ref,
               wf1_ref, bf1_ref, wf2_ref, bf2_ref, g2_ref, be2_ref,
               out_ref):
    p = p_ref[...]
    wv = jnp.dot(p[0] + p[1], pt_ref[...],
                 preferred_element_type=jnp.float32)
    z = z_ref[...]
    z8 = z[0] + z[1]
    zr = jnp.dot(z8, sel_ref[...], preferred_element_type=jnp.float32)
    ha = h_ref[...] + wv / (zr + 1e-6)
    hn = ha * (g1_ref[...] * INV_BN) + be1_ref[...]
    ff = jnp.maximum(
        jnp.dot(hn, wf1_ref[...], preferred_element_type=jnp.float32)
        + bf1_ref[...], 0.0)
    ff = jnp.dot(ff, wf2_ref[...], preferred_element_type=jnp.float32) + bf2_ref[...]
    out_ref[...] = (hn + ff) * (g2_ref[...] * INV_BN) + be2_ref[...]


def _run_post(pacc, z, h, pt, sel, g1, be1, wf1, bf1, wf2, bf2, g2, be2):
    blk = 1000
    full = lambda shape: pl.BlockSpec(shape, lambda i: tuple(0 for _ in shape))
    return pl.pallas_call(
        _post_body,
        grid=(N // blk,),
        in_specs=[
            pl.BlockSpec((2, blk, D), lambda i: (0, i, 0)),
            pl.BlockSpec((2, blk, H), lambda i: (0, i, 0)),
            pl.BlockSpec((blk, D), lambda i: (i, 0)),
            full((D, D)),
            full((H, D)),
            full((1, D)),
            full((1, D)),
            full((D, 2 * D)),
            full((1, 2 * D)),
            full((2 * D, D)),
            full((1, D)),
            full((1, D)),
            full((1, D)),
        ],
        out_specs=pl.BlockSpec((blk, D), lambda i: (i, 0)),
        out_shape=jax.ShapeDtypeStruct((N, D), jnp.float32),
    )(pacc, z, h, pt, sel, g1, be1, wf1, bf1, wf2, bf2, g2, be2)


def kernel(h, edge_index, edge_attr, WQ, WK, WV, WE, WEb, bEb, g1, be1,
           Wf1, bf1, Wf2, bf2, g2, be2):
    src = edge_index[0].astype(jnp.int32)
    dst = edge_index[1].astype(jnp.int32)

    p = jnp.arange(D, dtype=jnp.int32)
    r = p % 32
    colperm = 32 * (p // 32) + jnp.where(r < 16, 2 * r, 2 * (r - 16) + 1)
    pt = (jnp.arange(D)[None, :] == colperm[:, None]).astype(jnp.float32)

    wqp = jnp.concatenate([WQ, jnp.zeros((D, D), jnp.float32)], axis=1)
    wkv = jnp.concatenate([WK, WV], axis=1)
    wcat = jnp.concatenate(
        [WE[:, colperm], WEb, jnp.zeros((16, 8), jnp.float32)], axis=1)
    bcat = jnp.concatenate(
        [jnp.zeros((D,), jnp.float32), bEb, jnp.zeros((8,), jnp.float32)]
    ).reshape(1, W144)

    q, kv = _run_qkv(h, wqp, wkv)
    eeeb = _run_ee(edge_attr, wcat, bcat)
    kv32 = lax.bitcast_convert_type(kv.reshape(N, D, 2), jnp.int32)
    q32 = lax.bitcast_convert_type(q.reshape(N, D, 2), jnp.int32)
    pacc = _sc_edge(kv32, q32, eeeb, src, dst)

    z = pacc[:, ZBASE:ZBASE + (N * H) // D, :].reshape(2, N, H)

    sel = jnp.kron(jnp.eye(H, dtype=jnp.float32),
                   jnp.ones((1, DH), jnp.float32))
    h_out = _run_post(pacc, z, h, pt, sel,
                      g1.reshape(1, D), be1.reshape(1, D),
                      Wf1, bf1.reshape(1, 2 * D),
                      Wf2, bf2.reshape(1, D),
                      g2.reshape(1, D), be2.reshape(1, D))
    return (h_out, edge_attr)
